# baseline (speedup 1.0000x reference)
import functools

import jax
import jax.numpy as jnp
from jax import lax
from jax.experimental import pallas as pl
from jax.experimental.pallas import tpu as pltpu
from jax.experimental.pallas import tpu_sc as plsc

NC = 2
NS = 16
L = 16
NW = NC * NS
B = 16384
D = 16
BPW = B // NW
NCHUNK = 4
CW = BPW // NCHUNK
G = BPW // L


def _gmf_body(uidx_hbm, iidx_hbm, u1_hbm, i1_hbm, u2_hbm, i2_hbm, wb_hbm,
              out_hbm,
              uidx_v, iidx_v, u1_v, i1_v, u2_v, i2_v, wb_v, out_v, sem):
    wid = lax.axis_index("s") * NC + lax.axis_index("c")

    pltpu.sync_copy(wb_hbm, wb_v)
    pltpu.sync_copy(uidx_hbm.at[wid], uidx_v)
    pltpu.sync_copy(iidx_hbm.at[wid], iidx_v)

    copies = []
    for j in range(NCHUNK):
        rows = pl.ds(j * CW, CW)
        copies.append(pltpu.async_copy(u1_hbm.at[uidx_v.at[j]], u1_v.at[rows], sem))
        copies.append(pltpu.async_copy(i1_hbm.at[iidx_v.at[j]], i1_v.at[rows], sem))
        copies.append(pltpu.async_copy(u2_hbm.at[uidx_v.at[j]], u2_v.at[rows], sem))
        copies.append(pltpu.async_copy(i2_hbm.at[iidx_v.at[j]], i2_v.at[rows], sem))
    for c in copies:
        c.wait()

    w1 = wb_v[pl.ds(0, L)]
    w2 = wb_v[pl.ds(L, L)]
    bv = wb_v[pl.ds(2 * L, L)]
    lane = lax.iota(jnp.int32, L)

    def group(g, carry):
        base = g * L
        res = jnp.zeros((L,), jnp.float32)
        for k in range(L):
            r = base + k
            s = u1_v[r] * i1_v[r] * w1 + u2_v[r] * i2_v[r] * w2
            res = jnp.where(lane == k, jnp.sum(s), res)
        out_v[pl.ds(base, L)] = jnp.maximum(res + bv, 0.0)
        return carry

    lax.fori_loop(0, G, group, 0)
    pltpu.sync_copy(out_v, out_hbm.at[pl.ds(wid * BPW, BPW)])


@jax.jit
def kernel(user_inputs, item_inputs, U1, I1, U2, I2, W, b):
    wb = jnp.concatenate([W.reshape(2 * D), jnp.broadcast_to(b, (L,))])
    uidx = user_inputs.reshape(NW, NCHUNK, CW)
    iidx = item_inputs.reshape(NW, NCHUNK, CW)
    mesh = plsc.VectorSubcoreMesh(core_axis_name="c", subcore_axis_name="s",
                                  num_cores=NC, num_subcores=NS)
    run = pl.kernel(
        _gmf_body,
        out_type=jax.ShapeDtypeStruct((B,), jnp.float32),
        mesh=mesh,
        compiler_params=pltpu.CompilerParams(needs_layout_passes=False,
                                             use_tc_tiling_on_sc=False),
        scratch_types=[
            pltpu.VMEM((NCHUNK, CW), jnp.int32),
            pltpu.VMEM((NCHUNK, CW), jnp.int32),
            pltpu.VMEM((BPW, D), jnp.float32),
            pltpu.VMEM((BPW, D), jnp.float32),
            pltpu.VMEM((BPW, D), jnp.float32),
            pltpu.VMEM((BPW, D), jnp.float32),
            pltpu.VMEM((3 * L,), jnp.float32),
            pltpu.VMEM((BPW,), jnp.float32),
            pltpu.SemaphoreType.DMA,
        ],
    )
    out = run(uidx, iidx, U1, I1, U2, I2, wb)
    return out.reshape(B, 1)


# speedup vs baseline: 1.0006x; 1.0006x over previous
import jax
import jax.numpy as jnp
from jax import lax
from jax.experimental import pallas as pl
from jax.experimental.pallas import tpu as pltpu
from jax.experimental.pallas import tpu_sc as plsc

NC = 2
NS = 16
L = 16
NW = NC * NS
B = 16384
D = 16
BPW = B // NW
NCHUNK = 4
CW = BPW // NCHUNK
G = BPW // L


def _gmf_body(uidx_hbm, iidx_hbm, u1_hbm, i1_hbm, u2_hbm, i2_hbm, wb_hbm,
              out_hbm,
              uidx_v, iidx_v, u1_v, i1_v, u2_v, i2_v, wb_v, out_v, sem):
    wid = lax.axis_index("s") * NC + lax.axis_index("c")

    pltpu.sync_copy(wb_hbm, wb_v)
    pltpu.sync_copy(uidx_hbm.at[wid], uidx_v)
    pltpu.sync_copy(iidx_hbm.at[wid], iidx_v)

    copies = []
    for j in range(NCHUNK):
        rows = pl.ds(j * CW, CW)
        copies.append(pltpu.async_copy(u1_hbm.at[uidx_v.at[j]], u1_v.at[rows], sem))
        copies.append(pltpu.async_copy(i1_hbm.at[iidx_v.at[j]], i1_v.at[rows], sem))
        copies.append(pltpu.async_copy(u2_hbm.at[uidx_v.at[j]], u2_v.at[rows], sem))
        copies.append(pltpu.async_copy(i2_hbm.at[iidx_v.at[j]], i2_v.at[rows], sem))
    for c in copies:
        c.wait()

    w1 = wb_v[pl.ds(0, L)]
    w2 = wb_v[pl.ds(L, L)]
    bv = wb_v[pl.ds(2 * L, L)]
    lane = lax.iota(jnp.int32, L)

    def group(g, carry):
        base = g * L
        res = jnp.zeros((L,), jnp.float32)
        for k in range(L):
            r = base + k
            s = u1_v[r] * i1_v[r] * w1 + u2_v[r] * i2_v[r] * w2
            res = jnp.where(lane == k, jnp.sum(s), res)
        out_v[pl.ds(base, L)] = jnp.maximum(res + bv, 0.0)
        return carry

    lax.fori_loop(0, G, group, 0)
    pltpu.sync_copy(out_v, out_hbm.at[pl.ds(wid * BPW, BPW)])


@jax.jit
def kernel(user_inputs, item_inputs, U1, I1, U2, I2, W, b):
    wb = jnp.concatenate([W.reshape(2 * D), jnp.broadcast_to(b, (L,))])
    uidx = user_inputs.reshape(NW, NCHUNK, CW)
    iidx = item_inputs.reshape(NW, NCHUNK, CW)
    mesh = plsc.VectorSubcoreMesh(core_axis_name="c", subcore_axis_name="s",
                                  num_cores=NC, num_subcores=NS)
    run = pl.kernel(
        _gmf_body,
        out_type=jax.ShapeDtypeStruct((B,), jnp.float32),
        mesh=mesh,
        compiler_params=pltpu.CompilerParams(needs_layout_passes=False,
                                             use_tc_tiling_on_sc=False),
        scratch_types=[
            pltpu.VMEM((NCHUNK, CW), jnp.int32),
            pltpu.VMEM((NCHUNK, CW), jnp.int32),
            pltpu.VMEM((BPW, D), jnp.float32),
            pltpu.VMEM((BPW, D), jnp.float32),
            pltpu.VMEM((BPW, D), jnp.float32),
            pltpu.VMEM((BPW, D), jnp.float32),
            pltpu.VMEM((3 * L,), jnp.float32),
            pltpu.VMEM((BPW,), jnp.float32),
            pltpu.SemaphoreType.DMA,
        ],
    )
    out = run(uidx, iidx, U1, I1, U2, I2, wb)
    return out.reshape(B, 1)
